# baseline (speedup 1.0000x reference)
import functools

import jax
import jax.numpy as jnp
from jax import lax
from jax.experimental import pallas as pl
from jax.experimental.pallas import tpu as pltpu
from jax.experimental.pallas import tpu_sc as plsc

_NBUF = 2


def _make_lookup(B: int, D: int, NC: int, NS: int, C: int):
    NW = NC * NS
    b_per_w = B // NW
    n_chunks = b_per_w // C
    n_vecs = D // 16
    assert n_chunks % _NBUF == 0 and n_chunks >= 2 * _NBUF
    mesh = plsc.VectorSubcoreMesh(core_axis_name="c", subcore_axis_name="s")

    @functools.partial(
        pl.kernel,
        mesh=mesh,
        out_type=jax.ShapeDtypeStruct((B, D), jnp.float32),
        scratch_types=[
            pltpu.VMEM((b_per_w + 16,), jnp.int32),
            pltpu.VMEM((16, D), jnp.float32),
        ] + [pltpu.VMEM((C, D), jnp.float32) for _ in range(_NBUF)]
          + [pltpu.SemaphoreType.DMA for _ in range(_NBUF)],
    )
    def lookup_kernel(table_hbm, idx_hbm, out_hbm, idx_v, table_v,
                      *bufs_sems):
        rows = bufs_sems[:_NBUF]
        semo = bufs_sems[_NBUF:]
        wid = lax.axis_index("s") * NC + lax.axis_index("c")
        base = wid * b_per_w

        tcopy = pltpu.async_copy(table_hbm, table_v, semo[0])
        icopy = pltpu.async_copy(idx_hbm.at[pl.ds(base, b_per_w)],
                                 idx_v.at[pl.ds(0, b_per_w)], semo[1])
        tcopy.wait()
        icopy.wait()

        def fill(g, b, lo=0, nr=C):
            buf = rows[b]

            @plsc.parallel_loop(lo, lo + nr, unroll=2)
            def row(j):
                r = idx_v[pl.ds(g * C + j, 16)][0]
                vals = [table_v[r, pl.ds(16 * d, 16)]
                        for d in range(n_vecs)]
                for d in range(n_vecs):
                    buf[j, pl.ds(16 * d, 16)] = vals[d]

        def start_out(g, b, lo=0, nr=C):
            pltpu.async_copy(rows[b].at[pl.ds(lo, nr)],
                             out_hbm.at[pl.ds(base + g * C + lo, nr)],
                             semo[b])

        def wait_out(g, b):
            pltpu.make_async_copy(rows[b],
                                  out_hbm.at[pl.ds(base + g * C, C)],
                                  semo[b]).wait()

        Q = C // 4
        for q in range(4):
            fill(0, 0, q * Q, Q)
            start_out(0, 0, q * Q, Q)
        fill(1, 1)
        start_out(1, 1)

        def body(t, carry):
            for b in range(_NBUF):
                g = _NBUF * t + b
                wait_out(g, b)
                fill(g, b)
                start_out(g, b)
            return carry

        lax.fori_loop(1, n_chunks // _NBUF - 1, body, 0)

        gl = n_chunks - 2
        wait_out(gl, 0)
        fill(gl, 0)
        start_out(gl, 0)
        wait_out(gl + 1, 1)
        for q in range(4):
            fill(gl + 1, 1, q * Q, Q)
            start_out(gl + 1, 1, q * Q, Q)

        wait_out(gl, 0)
        wait_out(gl + 1, 1)

    return lookup_kernel


def kernel(x, table):
    B = x.shape[0] * x.shape[1]
    D = table.shape[1]
    info = plsc.get_sparse_core_info()
    NC, NS = info.num_cores, info.num_subcores
    lookup = _make_lookup(B, D, NC, NS, C=64)
    out = lookup(table, x.reshape(B).astype(jnp.int32))
    return out.reshape(x.shape[0], x.shape[1], D)


# speedup vs baseline: 1.0854x; 1.0854x over previous
import functools

import jax
import jax.numpy as jnp
from jax import lax
from jax.experimental import pallas as pl
from jax.experimental.pallas import tpu as pltpu
from jax.experimental.pallas import tpu_sc as plsc

_NBUF = 2


def _make_lookup(B: int, D: int, NC: int, NS: int, C: int):
    NW = NC * NS
    b_per_w = B // NW
    n_chunks = b_per_w // C
    n_vecs = D // 16
    assert n_chunks % _NBUF == 0 and n_chunks >= 2 * _NBUF
    mesh = plsc.VectorSubcoreMesh(core_axis_name="c", subcore_axis_name="s")

    @functools.partial(
        pl.kernel,
        mesh=mesh,
        out_type=jax.ShapeDtypeStruct((B, D), jnp.float32),
        scratch_types=[
            pltpu.VMEM((b_per_w + 16,), jnp.int32),
            pltpu.VMEM((16, D), jnp.float32),
        ] + [pltpu.VMEM((C, D), jnp.float32) for _ in range(_NBUF)]
          + [pltpu.SemaphoreType.DMA for _ in range(_NBUF)],
    )
    def lookup_kernel(table_hbm, idx_hbm, out_hbm, idx_v, table_v,
                      *bufs_sems):
        rows = bufs_sems[:_NBUF]
        semo = bufs_sems[_NBUF:]
        wid = lax.axis_index("s") * NC + lax.axis_index("c")
        base = wid * b_per_w

        tcopy = pltpu.async_copy(table_hbm, table_v, semo[0])
        icopy = pltpu.async_copy(idx_hbm.at[pl.ds(base, b_per_w)],
                                 idx_v.at[pl.ds(0, b_per_w)], semo[1])
        tcopy.wait()
        icopy.wait()

        def fill(g, b):
            buf = rows[b]

            @plsc.parallel_loop(0, C, unroll=2)
            def row(j):
                r = idx_v[pl.ds(g * C + j, 16)][0]
                vals = [table_v[r, pl.ds(16 * d, 16)]
                        for d in range(n_vecs)]
                for d in range(n_vecs):
                    buf[j, pl.ds(16 * d, 16)] = vals[d]

        def start_out(g, b):
            pltpu.async_copy(rows[b], out_hbm.at[pl.ds(base + g * C, C)],
                             semo[b])

        def wait_out(g, b):
            pltpu.make_async_copy(rows[b],
                                  out_hbm.at[pl.ds(base + g * C, C)],
                                  semo[b]).wait()

        def body(t, carry):
            for b in range(_NBUF):
                g = _NBUF * t + b

                @pl.when(g >= _NBUF)
                def _():
                    wait_out(g, b)

                fill(g, b)
                start_out(g, b)
            return carry

        lax.fori_loop(0, n_chunks // _NBUF, body, 0)

        for b in range(_NBUF):
            wait_out(n_chunks - _NBUF + b, b)

    return lookup_kernel


def kernel(x, table):
    B = x.shape[0] * x.shape[1]
    D = table.shape[1]
    info = plsc.get_sparse_core_info()
    NC, NS = info.num_cores, info.num_subcores
    lookup = _make_lookup(B, D, NC, NS, C=64)
    out = lookup(table, x.reshape(B).astype(jnp.int32))
    return out.reshape(x.shape[0], x.shape[1], D)
